# baseline (speedup 1.0000x reference)
import jax
import jax.numpy as jnp
from jax import lax
from jax.experimental import pallas as pl
from jax.experimental.pallas import tpu as pltpu
from jax.experimental.pallas import tpu_sc as plsc

N = 10000
E = 320000
F_IN = 7409
D = 64
NUM_SEG = 2000
NUM_CLASSES = 7

BK = 1024
K_STEPS = 8
K_PAD = BK * K_STEPS
ER = E // 128
NR = 79
N_PAD = NR * 128
RPT = N_PAD // 16
SEG_PAD = 2048
SPT = SEG_PAD // 16
NSUP = 20
E2R = (NSUP + 1) * 128
NIR = 128

_MESH = plsc.VectorSubcoreMesh(core_axis_name="c", subcore_axis_name="s")


def _hist_body(dst1, dst2, i1, i2, ones_h, zN, zS,
               deg1_o, deg2_o, cnt1_o, cnt2_o,
               ones_v, idx0, idx1, deg_sh, cnt_sh, sem_i, sem_s):
    cid = lax.axis_index("c")
    sid = lax.axis_index("s")

    @pl.when(sid == 0)
    def _():
        pltpu.sync_copy(zN, deg_sh)
        pltpu.sync_copy(zS, cnt_sh)

    pltpu.sync_copy(ones_h, ones_v)
    plsc.subcore_barrier()

    def scatter_ones(idx2d, nsup, acc_sh):
        pltpu.sync_copy(idx2d.at[pl.ds(sid * 8, 8), :], idx0)

        def sup(u, cur, nxt):
            h_i = pltpu.async_copy(
                idx2d.at[pl.ds((u + 1) * 128 + sid * 8, 8), :], nxt, sem_i)
            hs_ = [pltpu.async_copy(ones_v, acc_sh.at[cur.at[j]], sem_s,
                                    add=True) for j in range(8)]
            for h in hs_:
                h.wait()
            h_i.wait()

        def pair(p, c):
            sup(2 * p, idx0, idx1)
            sup(2 * p + 1, idx1, idx0)
            return c

        lax.fori_loop(0, nsup // 2, pair, 0)

    def scatter_ones_once(idx2d, acc_sh):
        pltpu.sync_copy(idx2d.at[pl.ds(sid * 8, 8), :], idx0)
        hs_ = [pltpu.async_copy(ones_v, acc_sh.at[idx0.at[j]], sem_s,
                                add=True) for j in range(8)]
        for h in hs_:
            h.wait()

    @pl.when(cid == 0)
    def _():
        scatter_ones(dst1, NSUP, deg_sh)
        scatter_ones_once(i1, cnt_sh)

    @pl.when(cid == 1)
    def _():
        scatter_ones(dst2, NSUP, deg_sh)
        scatter_ones_once(i2, cnt_sh)

    plsc.subcore_barrier()

    @pl.when(jnp.logical_and(sid == 0, cid == 0))
    def _():
        pltpu.sync_copy(deg_sh, deg1_o)
        pltpu.sync_copy(cnt_sh, cnt1_o)

    @pl.when(jnp.logical_and(sid == 0, cid == 1))
    def _():
        pltpu.sync_copy(deg_sh, deg2_o)
        pltpu.sync_copy(cnt_sh, cnt2_o)


_hist = pl.kernel(
    _hist_body,
    out_type=[
        jax.ShapeDtypeStruct((N_PAD,), jnp.float32),
        jax.ShapeDtypeStruct((N_PAD,), jnp.float32),
        jax.ShapeDtypeStruct((SEG_PAD,), jnp.float32),
        jax.ShapeDtypeStruct((SEG_PAD,), jnp.float32),
    ],
    mesh=_MESH,
    compiler_params=pltpu.CompilerParams(use_tc_tiling_on_sc=False),
    scratch_types=[
        pltpu.VMEM((128,), jnp.float32),
        pltpu.VMEM((8, 128), jnp.int32),
        pltpu.VMEM((8, 128), jnp.int32),
        pltpu.VMEM_SHARED((N_PAD,), jnp.float32),
        pltpu.VMEM_SHARED((SEG_PAD,), jnp.float32),
        pltpu.SemaphoreType.DMA,
        pltpu.SemaphoreType.DMA,
    ],
)


def _mp_body(hs1, hs2, src1, dst1, src2, dst2, z64,
             agg1_o, agg2_o,
             sidx0, didx0, rows0, sidx1, didx1, rows1,
             acc_sh, sem_i, sem_g, sem_s):
    cid = lax.axis_index("c")
    sid = lax.axis_index("s")
    r0 = sid * RPT
    pltpu.sync_copy(z64.at[pl.ds(r0, RPT), :], acc_sh.at[pl.ds(r0, RPT), :])
    plsc.subcore_barrier()

    def edge_pass(hs, src2d, dst2d):
        rows = (rows0, rows1)
        pltpu.sync_copy(src2d.at[pl.ds(sid * 8, 8), :], sidx0)
        pltpu.sync_copy(dst2d.at[pl.ds(sid * 8, 8), :], didx0)
        pltpu.async_copy(hs.at[sidx0.at[0]], rows0, sem_g).wait()

        def sup(u, cur, nxt):
            sidx_c, didx_c = cur
            sidx_n, didx_n = nxt
            nbase = (u + 1) * 128 + sid * 8
            h_i1 = pltpu.async_copy(src2d.at[pl.ds(nbase, 8), :], sidx_n,
                                    sem_i)
            h_i2 = pltpu.async_copy(dst2d.at[pl.ds(nbase, 8), :], didx_n,
                                    sem_i)
            for j in range(8):
                rc = rows[j % 2]
                rn = rows[1 - j % 2]
                h_s = pltpu.async_copy(rc, acc_sh.at[didx_c.at[j]], sem_s,
                                       add=True)
                if j < 7:
                    pltpu.async_copy(hs.at[sidx_c.at[j + 1]], rn,
                                     sem_g).wait()
                else:
                    h_i1.wait()
                    h_i2.wait()
                    pltpu.async_copy(hs.at[sidx_n.at[0]], rn, sem_g).wait()
                h_s.wait()

        def pair(p, c):
            sup(2 * p, (sidx0, didx0), (sidx1, didx1))
            sup(2 * p + 1, (sidx1, didx1), (sidx0, didx0))
            return c

        lax.fori_loop(0, NSUP // 2, pair, 0)

    @pl.when(cid == 0)
    def _():
        edge_pass(hs1, src1, dst1)

    @pl.when(cid == 1)
    def _():
        edge_pass(hs2, src2, dst2)

    plsc.subcore_barrier()

    @pl.when(cid == 0)
    def _():
        pltpu.sync_copy(acc_sh.at[pl.ds(r0, RPT), :],
                        agg1_o.at[pl.ds(r0, RPT), :])

    @pl.when(cid == 1)
    def _():
        pltpu.sync_copy(acc_sh.at[pl.ds(r0, RPT), :],
                        agg2_o.at[pl.ds(r0, RPT), :])


_mp = pl.kernel(
    _mp_body,
    out_type=[jax.ShapeDtypeStruct((N_PAD, D), jnp.float32)] * 2,
    mesh=_MESH,
    compiler_params=pltpu.CompilerParams(use_tc_tiling_on_sc=False),
    scratch_types=[
        pltpu.VMEM((8, 128), jnp.int32),
        pltpu.VMEM((8, 128), jnp.int32),
        pltpu.VMEM((128, D), jnp.float32),
        pltpu.VMEM((8, 128), jnp.int32),
        pltpu.VMEM((8, 128), jnp.int32),
        pltpu.VMEM((128, D), jnp.float32),
        pltpu.VMEM_SHARED((N_PAD, D), jnp.float32),
        pltpu.SemaphoreType.DMA,
        pltpu.SemaphoreType.DMA,
        pltpu.SemaphoreType.DMA,
    ],
)


def _pool_body(hfin, i1, i2, zS64, s1_o, s2_o, cidx_v, rows_v, acc_sh,
               sem_g, sem_s):
    cid = lax.axis_index("c")
    sid = lax.axis_index("s")
    r0 = sid * SPT
    pltpu.sync_copy(zS64.at[pl.ds(r0, SPT), :], acc_sh.at[pl.ds(r0, SPT), :])
    plsc.subcore_barrier()

    def pool_pass(idx2d):
        pltpu.sync_copy(idx2d.at[pl.ds(sid * 8, 8), :], cidx_v)
        for j in range(8):
            row = sid * 8 + j

            @pl.when(row < NR)
            def _():
                pltpu.async_copy(hfin.at[pl.ds(row * 128, 128), :], rows_v,
                                 sem_g).wait()
                pltpu.async_copy(rows_v, acc_sh.at[cidx_v.at[j]], sem_s,
                                 add=True).wait()

    @pl.when(cid == 0)
    def _():
        pool_pass(i1)

    @pl.when(cid == 1)
    def _():
        pool_pass(i2)

    plsc.subcore_barrier()

    @pl.when(cid == 0)
    def _():
        pltpu.sync_copy(acc_sh.at[pl.ds(r0, SPT), :],
                        s1_o.at[pl.ds(r0, SPT), :])

    @pl.when(cid == 1)
    def _():
        pltpu.sync_copy(acc_sh.at[pl.ds(r0, SPT), :],
                        s2_o.at[pl.ds(r0, SPT), :])


_pool = pl.kernel(
    _pool_body,
    out_type=[jax.ShapeDtypeStruct((SEG_PAD, D), jnp.float32)] * 2,
    mesh=_MESH,
    compiler_params=pltpu.CompilerParams(use_tc_tiling_on_sc=False),
    scratch_types=[
        pltpu.VMEM((8, 128), jnp.int32),
        pltpu.VMEM((128, D), jnp.float32),
        pltpu.VMEM_SHARED((SEG_PAD, D), jnp.float32),
        pltpu.SemaphoreType.DMA,
        pltpu.SemaphoreType.DMA,
    ],
)


BM = 632


def _t1_body(x_ref, w_ref, h1_ref, h2_ref, acc_ref):
    k = pl.program_id(1)

    @pl.when(k == 0)
    def _():
        acc_ref[...] = jnp.zeros_like(acc_ref)

    @pl.when(k < K_STEPS - 1)
    def _():
        acc_ref[...] += jnp.dot(x_ref[...], w_ref[...],
                                preferred_element_type=jnp.float32)

    @pl.when(k == K_STEPS - 1)
    def _():
        col = lax.broadcasted_iota(jnp.int32, (BM, BK), 1) + k * BK
        xb = jnp.where(col < F_IN, x_ref[...], 0.0)
        h = acc_ref[...] + jnp.dot(xb, w_ref[...],
                                   preferred_element_type=jnp.float32)
        h1_ref[...] = h[:, :D]
        h2_ref[...] = h[:, D:]


_t1 = pl.pallas_call(
    _t1_body,
    grid=(N_PAD // BM, K_STEPS),
    in_specs=[
        pl.BlockSpec((BM, BK), lambda i, k: (i, k)),
        pl.BlockSpec((BK, 2 * D), lambda i, k: (k, 0)),
    ],
    out_specs=[pl.BlockSpec((BM, D), lambda i, k: (i, 0))] * 2,
    out_shape=[jax.ShapeDtypeStruct((N_PAD, D), jnp.float32)] * 2,
    scratch_shapes=[pltpu.VMEM((BM, 2 * D), jnp.float32)],
    compiler_params=pltpu.CompilerParams(
        dimension_semantics=("parallel", "arbitrary")),
)


def _t1b_body(h1, h2, deg1, deg2, hs1_ref, hs2_ref):
    hs1_ref[...] = h1[...] * lax.rsqrt(deg1[...] + 1.0)
    hs2_ref[...] = h2[...] * lax.rsqrt(deg2[...] + 1.0)


_t1b = pl.pallas_call(
    _t1b_body,
    grid=(N_PAD // BM,),
    in_specs=[
        pl.BlockSpec((BM, D), lambda i: (i, 0)),
        pl.BlockSpec((BM, D), lambda i: (i, 0)),
        pl.BlockSpec((BM, 1), lambda i: (i, 0)),
        pl.BlockSpec((BM, 1), lambda i: (i, 0)),
    ],
    out_specs=[pl.BlockSpec((BM, D), lambda i: (i, 0))] * 2,
    out_shape=[jax.ShapeDtypeStruct((N_PAD, D), jnp.float32)] * 2,
)


def _t2_body(agg1, agg2, hs1, hs2, deg1, deg2, b11, b12,
             M1a1, M1a2, m1ab, M1b, m1bb, W21, W22, hs1p, hs2p):
    d1 = lax.rsqrt(deg1[...] + 1.0)
    d2 = lax.rsqrt(deg2[...] + 1.0)
    x1 = jax.nn.relu(d1 * (agg1[...] + hs1[...]) + b11[...])
    x2 = jax.nn.relu(d2 * (agg2[...] + hs2[...]) + b12[...])
    t = jax.nn.relu(
        jnp.dot(x1, M1a1[...], preferred_element_type=jnp.float32)
        + jnp.dot(x2, M1a2[...], preferred_element_type=jnp.float32)
        + m1ab[...])
    h2 = jnp.dot(t, M1b[...], preferred_element_type=jnp.float32) + m1bb[...]
    hs1p[...] = jnp.dot(h2, W21[...], preferred_element_type=jnp.float32) * d1
    hs2p[...] = jnp.dot(h2, W22[...], preferred_element_type=jnp.float32) * d2


def _row_spec(w):
    return pl.BlockSpec((BM, w), lambda i: (i, 0))


def _full_spec(a, b):
    return pl.BlockSpec((a, b), lambda i: (0, 0))


_t2 = pl.pallas_call(
    _t2_body,
    grid=(N_PAD // BM,),
    in_specs=[
        _row_spec(D), _row_spec(D), _row_spec(D), _row_spec(D),
        _row_spec(1), _row_spec(1),
        _full_spec(1, D), _full_spec(1, D),
        _full_spec(D, D), _full_spec(D, D), _full_spec(1, D),
        _full_spec(D, D), _full_spec(1, D),
        _full_spec(D, D), _full_spec(D, D),
    ],
    out_specs=[_row_spec(D)] * 2,
    out_shape=[jax.ShapeDtypeStruct((N_PAD, D), jnp.float32)] * 2,
)


def _t3_body(agg1, agg2, hs1, hs2, deg1, deg2, b21, b22,
             M2a1, M2a2, m2ab, M2b, m2bb, hfin):
    i = pl.program_id(0)
    d1 = lax.rsqrt(deg1[...] + 1.0)
    d2 = lax.rsqrt(deg2[...] + 1.0)
    x1 = jax.nn.relu(d1 * (agg1[...] + hs1[...]) + b21[...])
    x2 = jax.nn.relu(d2 * (agg2[...] + hs2[...]) + b22[...])
    t = jax.nn.relu(
        jnp.dot(x1, M2a1[...], preferred_element_type=jnp.float32)
        + jnp.dot(x2, M2a2[...], preferred_element_type=jnp.float32)
        + m2ab[...])
    h = jnp.dot(t, M2b[...], preferred_element_type=jnp.float32) + m2bb[...]
    row = lax.broadcasted_iota(jnp.int32, (BM, D), 0) + i * BM
    hfin[...] = jnp.where(row < N, h, 0.0)


_t3 = pl.pallas_call(
    _t3_body,
    grid=(N_PAD // BM,),
    in_specs=[
        _row_spec(D), _row_spec(D), _row_spec(D), _row_spec(D),
        _row_spec(1), _row_spec(1),
        _full_spec(1, D), _full_spec(1, D),
        _full_spec(D, D), _full_spec(D, D), _full_spec(1, D),
        _full_spec(D, D), _full_spec(1, D),
    ],
    out_specs=_row_spec(D),
    out_shape=jax.ShapeDtypeStruct((N_PAD, D), jnp.float32),
)

LG = 128


def _t4_body(S1, S2, cnt1, cnt2, Ma1, Ma2, mab, Mb_p, mbb_p, out):
    g1 = S1[...] / jnp.maximum(cnt1[...], 1.0)
    g2 = S2[...] / jnp.maximum(cnt2[...], 1.0)
    t = jax.nn.relu(
        jnp.dot(g1, Ma1[...], preferred_element_type=jnp.float32)
        + jnp.dot(g2, Ma2[...], preferred_element_type=jnp.float32)
        + mab[...])
    lg = jnp.dot(t, Mb_p[...], preferred_element_type=jnp.float32) + mbb_p[...]
    mx = jnp.max(lg, axis=1, keepdims=True)
    e = jnp.exp(lg - mx)
    out[...] = lg - mx - jnp.log(jnp.sum(e, axis=1, keepdims=True))


_t4 = pl.pallas_call(
    _t4_body,
    grid=(1,),
    in_specs=[
        pl.BlockSpec((NUM_SEG, D), lambda i: (0, 0)),
        pl.BlockSpec((NUM_SEG, D), lambda i: (0, 0)),
        pl.BlockSpec((NUM_SEG, 1), lambda i: (0, 0)),
        pl.BlockSpec((NUM_SEG, 1), lambda i: (0, 0)),
        _full_spec(D, D), _full_spec(D, D), _full_spec(1, D),
        _full_spec(D, LG), _full_spec(1, LG),
    ],
    out_specs=pl.BlockSpec((NUM_SEG, LG), lambda i: (0, 0)),
    out_shape=jax.ShapeDtypeStruct((NUM_SEG, LG), jnp.float32),
)


def kernel(x, edge_index_1, edge_index_2, index_1, index_2,
           W11, b11, W12, b12, M1a, m1ab, M1b, m1bb,
           W21, b21, W22, b22, M2a, m2ab, M2b, m2bb, Ma, mab, Mb, mbb):
    f32 = jnp.float32
    W_pad = (jnp.zeros((K_PAD, 2 * D), f32)
             .at[:F_IN, :D].set(W11).at[:F_IN, D:].set(W12))

    pad_e = jnp.full((E2R * 128 - E,), N, jnp.int32)

    def edges2d(e):
        return jnp.concatenate([e, pad_e]).reshape(E2R, 128)

    src1 = edges2d(edge_index_1[0])
    dst1 = edges2d(edge_index_1[1])
    src2 = edges2d(edge_index_2[0])
    dst2 = edges2d(edge_index_2[1])
    pad_i = jnp.full((NIR * 128 - N,), NUM_SEG, jnp.int32)
    i1 = jnp.concatenate([index_1, pad_i]).reshape(NIR, 128)
    i2 = jnp.concatenate([index_2, pad_i]).reshape(NIR, 128)

    ones128 = jnp.ones((128,), f32)
    zN = jnp.zeros((N_PAD,), f32)
    zS = jnp.zeros((SEG_PAD,), f32)
    z64 = jnp.zeros((N_PAD, D), f32)
    zS64 = jnp.zeros((SEG_PAD, D), f32)

    deg1, deg2, cnt1, cnt2 = _hist(dst1, dst2, i1, i2, ones128, zN, zS)
    deg1c = deg1[:N].reshape(N, 1)
    deg2c = deg2[:N].reshape(N, 1)

    h1, h2 = _t1(x, W_pad)
    hs1, hs2 = _t1b(h1, h2, deg1c, deg2c)
    agg1, agg2 = _mp(hs1, hs2, src1, dst1, src2, dst2, z64)
    hs1p, hs2p = _t2(agg1, agg2, hs1, hs2, deg1c, deg2c,
                     b11.reshape(1, D), b12.reshape(1, D),
                     M1a[:D], M1a[D:], m1ab.reshape(1, D),
                     M1b, m1bb.reshape(1, D), W21, W22)
    agg1p, agg2p = _mp(hs1p, hs2p, src1, dst1, src2, dst2, z64)
    hfin = _t3(agg1p, agg2p, hs1p, hs2p, deg1c, deg2c,
               b21.reshape(1, D), b22.reshape(1, D),
               M2a[:D], M2a[D:], m2ab.reshape(1, D),
               M2b, m2bb.reshape(1, D))
    S1, S2 = _pool(hfin, i1, i2, zS64)

    Mb_p = jnp.zeros((D, LG), f32).at[:, :NUM_CLASSES].set(Mb)
    mbb_p = jnp.full((1, LG), -1e30, f32).at[0, :NUM_CLASSES].set(mbb)
    out = _t4(S1[:NUM_SEG], S2[:NUM_SEG],
              cnt1[:NUM_SEG].reshape(NUM_SEG, 1),
              cnt2[:NUM_SEG].reshape(NUM_SEG, 1),
              Ma[:D], Ma[D:], mab.reshape(1, D), Mb_p, mbb_p)
    return out[:, :NUM_CLASSES]


# speedup vs baseline: 1.4962x; 1.4962x over previous
import jax
import jax.numpy as jnp
from jax import lax
from jax.experimental import pallas as pl
from jax.experimental.pallas import tpu as pltpu
from jax.experimental.pallas import tpu_sc as plsc

N = 10000
E = 320000
F_IN = 7409
D = 64
NUM_SEG = 2000
NUM_CLASSES = 7

BK = 1024
K_STEPS = 8
K_PAD = BK * K_STEPS
ER = E // 128
NR = 79
N_PAD = NR * 128
RPT = N_PAD // 16
SEG_PAD = 2048
SPT = SEG_PAD // 16
NSUP = 20
E2R = (NSUP + 1) * 128
NIR = 128

_MESH = plsc.VectorSubcoreMesh(core_axis_name="c", subcore_axis_name="s")


def _hist_body(dst1, dst2, i1, i2, ones_h, zN, zS,
               deg1_o, deg2_o, cnt1_o, cnt2_o,
               ones_v, idx0, idx1, deg_sh, cnt_sh, sem_i, sem_s):
    cid = lax.axis_index("c")
    sid = lax.axis_index("s")

    @pl.when(sid == 0)
    def _():
        pltpu.sync_copy(zN, deg_sh)
        pltpu.sync_copy(zS, cnt_sh)

    pltpu.sync_copy(ones_h, ones_v)
    plsc.subcore_barrier()

    def scatter_ones(idx2d, nsup, acc_sh):
        pltpu.sync_copy(idx2d.at[pl.ds(sid * 8, 8), :], idx0)

        def sup(u, cur, nxt):
            h_i = pltpu.async_copy(
                idx2d.at[pl.ds((u + 1) * 128 + sid * 8, 8), :], nxt, sem_i)
            hs_ = [pltpu.async_copy(ones_v, acc_sh.at[cur.at[j]], sem_s,
                                    add=True) for j in range(8)]
            for h in hs_:
                h.wait()
            h_i.wait()

        def pair(p, c):
            sup(2 * p, idx0, idx1)
            sup(2 * p + 1, idx1, idx0)
            return c

        lax.fori_loop(0, nsup // 2, pair, 0)

    def scatter_ones_once(idx2d, acc_sh):
        pltpu.sync_copy(idx2d.at[pl.ds(sid * 8, 8), :], idx0)
        hs_ = [pltpu.async_copy(ones_v, acc_sh.at[idx0.at[j]], sem_s,
                                add=True) for j in range(8)]
        for h in hs_:
            h.wait()

    @pl.when(cid == 0)
    def _():
        scatter_ones(dst1, NSUP, deg_sh)
        scatter_ones_once(i1, cnt_sh)

    @pl.when(cid == 1)
    def _():
        scatter_ones(dst2, NSUP, deg_sh)
        scatter_ones_once(i2, cnt_sh)

    plsc.subcore_barrier()

    @pl.when(jnp.logical_and(sid == 0, cid == 0))
    def _():
        pltpu.sync_copy(deg_sh, deg1_o)
        pltpu.sync_copy(cnt_sh, cnt1_o)

    @pl.when(jnp.logical_and(sid == 0, cid == 1))
    def _():
        pltpu.sync_copy(deg_sh, deg2_o)
        pltpu.sync_copy(cnt_sh, cnt2_o)


_hist = pl.kernel(
    _hist_body,
    out_type=[
        jax.ShapeDtypeStruct((N_PAD,), jnp.float32),
        jax.ShapeDtypeStruct((N_PAD,), jnp.float32),
        jax.ShapeDtypeStruct((SEG_PAD,), jnp.float32),
        jax.ShapeDtypeStruct((SEG_PAD,), jnp.float32),
    ],
    mesh=_MESH,
    compiler_params=pltpu.CompilerParams(use_tc_tiling_on_sc=False),
    scratch_types=[
        pltpu.VMEM((128,), jnp.float32),
        pltpu.VMEM((8, 128), jnp.int32),
        pltpu.VMEM((8, 128), jnp.int32),
        pltpu.VMEM_SHARED((N_PAD,), jnp.float32),
        pltpu.VMEM_SHARED((SEG_PAD,), jnp.float32),
        pltpu.SemaphoreType.DMA,
        pltpu.SemaphoreType.DMA,
    ],
)


def _mp_body(hs1, hs2, src1, dst1, src2, dst2, z64,
             agg1_o, agg2_o,
             sidx0, didx0, rows0, sidx1, didx1, rows1,
             acc_sh, sem_i, sem_g, sem_s):
    cid = lax.axis_index("c")
    sid = lax.axis_index("s")
    r0 = sid * RPT
    pltpu.sync_copy(z64.at[pl.ds(r0, RPT), :], acc_sh.at[pl.ds(r0, RPT), :])
    plsc.subcore_barrier()

    def edge_pass(hs, src2d, dst2d):
        rows = (rows0, rows1)
        pltpu.sync_copy(src2d.at[pl.ds(sid * 8, 8), :], sidx0)
        pltpu.sync_copy(dst2d.at[pl.ds(sid * 8, 8), :], didx0)
        pltpu.async_copy(hs.at[sidx0.at[0]], rows0, sem_g).wait()

        def sup(u, cur, nxt):
            sidx_c, didx_c = cur
            sidx_n, didx_n = nxt
            nbase = (u + 1) * 128 + sid * 8
            h_i1 = pltpu.async_copy(src2d.at[pl.ds(nbase, 8), :], sidx_n,
                                    sem_i)
            h_i2 = pltpu.async_copy(dst2d.at[pl.ds(nbase, 8), :], didx_n,
                                    sem_i)
            for j in range(8):
                rc = rows[j % 2]
                rn = rows[1 - j % 2]
                h_s = pltpu.async_copy(rc, acc_sh.at[didx_c.at[j]], sem_s,
                                       add=True)
                if j < 7:
                    pltpu.async_copy(hs.at[sidx_c.at[j + 1]], rn,
                                     sem_g).wait()
                else:
                    h_i1.wait()
                    h_i2.wait()
                    pltpu.async_copy(hs.at[sidx_n.at[0]], rn, sem_g).wait()
                h_s.wait()

        def pair(p, c):
            sup(2 * p, (sidx0, didx0), (sidx1, didx1))
            sup(2 * p + 1, (sidx1, didx1), (sidx0, didx0))
            return c

        lax.fori_loop(0, NSUP // 2, pair, 0)

    @pl.when(cid == 0)
    def _():
        edge_pass(hs1, src1, dst1)

    @pl.when(cid == 1)
    def _():
        edge_pass(hs2, src2, dst2)

    plsc.subcore_barrier()

    @pl.when(cid == 0)
    def _():
        pltpu.sync_copy(acc_sh.at[pl.ds(r0, RPT), :],
                        agg1_o.at[pl.ds(r0, RPT), :])

    @pl.when(cid == 1)
    def _():
        pltpu.sync_copy(acc_sh.at[pl.ds(r0, RPT), :],
                        agg2_o.at[pl.ds(r0, RPT), :])


_mp = pl.kernel(
    _mp_body,
    out_type=[jax.ShapeDtypeStruct((N_PAD, D), jnp.float32)] * 2,
    mesh=_MESH,
    compiler_params=pltpu.CompilerParams(use_tc_tiling_on_sc=False),
    scratch_types=[
        pltpu.VMEM((8, 128), jnp.int32),
        pltpu.VMEM((8, 128), jnp.int32),
        pltpu.VMEM((128, D), jnp.float32),
        pltpu.VMEM((8, 128), jnp.int32),
        pltpu.VMEM((8, 128), jnp.int32),
        pltpu.VMEM((128, D), jnp.float32),
        pltpu.VMEM_SHARED((N_PAD, D), jnp.float32),
        pltpu.SemaphoreType.DMA,
        pltpu.SemaphoreType.DMA,
        pltpu.SemaphoreType.DMA,
    ],
)


def _pool_body(hfin, i1, i2, zS64, s1_o, s2_o, cidx_v, rows_v, acc_sh,
               sem_g, sem_s):
    cid = lax.axis_index("c")
    sid = lax.axis_index("s")
    r0 = sid * SPT
    pltpu.sync_copy(zS64.at[pl.ds(r0, SPT), :], acc_sh.at[pl.ds(r0, SPT), :])
    plsc.subcore_barrier()

    def pool_pass(idx2d):
        pltpu.sync_copy(idx2d.at[pl.ds(sid * 8, 8), :], cidx_v)
        for j in range(8):
            row = sid * 8 + j

            @pl.when(row < NR)
            def _():
                pltpu.async_copy(hfin.at[pl.ds(row * 128, 128), :], rows_v,
                                 sem_g).wait()
                pltpu.async_copy(rows_v, acc_sh.at[cidx_v.at[j]], sem_s,
                                 add=True).wait()

    @pl.when(cid == 0)
    def _():
        pool_pass(i1)

    @pl.when(cid == 1)
    def _():
        pool_pass(i2)

    plsc.subcore_barrier()

    @pl.when(cid == 0)
    def _():
        pltpu.sync_copy(acc_sh.at[pl.ds(r0, SPT), :],
                        s1_o.at[pl.ds(r0, SPT), :])

    @pl.when(cid == 1)
    def _():
        pltpu.sync_copy(acc_sh.at[pl.ds(r0, SPT), :],
                        s2_o.at[pl.ds(r0, SPT), :])


_pool = pl.kernel(
    _pool_body,
    out_type=[jax.ShapeDtypeStruct((SEG_PAD, D), jnp.float32)] * 2,
    mesh=_MESH,
    compiler_params=pltpu.CompilerParams(use_tc_tiling_on_sc=False),
    scratch_types=[
        pltpu.VMEM((8, 128), jnp.int32),
        pltpu.VMEM((128, D), jnp.float32),
        pltpu.VMEM_SHARED((SEG_PAD, D), jnp.float32),
        pltpu.SemaphoreType.DMA,
        pltpu.SemaphoreType.DMA,
    ],
)


BM = 632


def _t1_body(x_ref, w_ref, h1_ref, h2_ref, acc_ref):
    k = pl.program_id(1)

    @pl.when(k == 0)
    def _():
        acc_ref[...] = jnp.zeros_like(acc_ref)

    @pl.when(k < K_STEPS - 1)
    def _():
        acc_ref[...] += jnp.dot(x_ref[...], w_ref[...],
                                preferred_element_type=jnp.float32)

    @pl.when(k == K_STEPS - 1)
    def _():
        col = lax.broadcasted_iota(jnp.int32, (BM, BK), 1) + k * BK
        xb = jnp.where(col < F_IN, x_ref[...], 0.0)
        h = acc_ref[...] + jnp.dot(xb, w_ref[...],
                                   preferred_element_type=jnp.float32)
        h1_ref[...] = h[:, :D]
        h2_ref[...] = h[:, D:]


_t1 = pl.pallas_call(
    _t1_body,
    grid=(N_PAD // BM, K_STEPS),
    in_specs=[
        pl.BlockSpec((BM, BK), lambda i, k: (i, k)),
        pl.BlockSpec((BK, 2 * D), lambda i, k: (k, 0)),
    ],
    out_specs=[pl.BlockSpec((BM, D), lambda i, k: (i, 0))] * 2,
    out_shape=[jax.ShapeDtypeStruct((N_PAD, D), jnp.float32)] * 2,
    scratch_shapes=[pltpu.VMEM((BM, 2 * D), jnp.float32)],
    compiler_params=pltpu.CompilerParams(
        dimension_semantics=("parallel", "arbitrary")),
)


def _t1b_body(h1, h2, deg1, deg2, hs1_ref, hs2_ref):
    hs1_ref[...] = h1[...] * lax.rsqrt(deg1[...] + 1.0)
    hs2_ref[...] = h2[...] * lax.rsqrt(deg2[...] + 1.0)


_t1b = pl.pallas_call(
    _t1b_body,
    grid=(N_PAD // BM,),
    in_specs=[
        pl.BlockSpec((BM, D), lambda i: (i, 0)),
        pl.BlockSpec((BM, D), lambda i: (i, 0)),
        pl.BlockSpec((BM, 1), lambda i: (i, 0)),
        pl.BlockSpec((BM, 1), lambda i: (i, 0)),
    ],
    out_specs=[pl.BlockSpec((BM, D), lambda i: (i, 0))] * 2,
    out_shape=[jax.ShapeDtypeStruct((N_PAD, D), jnp.float32)] * 2,
)


def _t2_body(agg1, agg2, hs1, hs2, deg1, deg2, b11, b12,
             M1a1, M1a2, m1ab, M1b, m1bb, W21, W22, hs1p, hs2p):
    d1 = lax.rsqrt(deg1[...] + 1.0)
    d2 = lax.rsqrt(deg2[...] + 1.0)
    x1 = jax.nn.relu(d1 * (agg1[...] + hs1[...]) + b11[...])
    x2 = jax.nn.relu(d2 * (agg2[...] + hs2[...]) + b12[...])
    t = jax.nn.relu(
        jnp.dot(x1, M1a1[...], preferred_element_type=jnp.float32)
        + jnp.dot(x2, M1a2[...], preferred_element_type=jnp.float32)
        + m1ab[...])
    h2 = jnp.dot(t, M1b[...], preferred_element_type=jnp.float32) + m1bb[...]
    hs1p[...] = jnp.dot(h2, W21[...], preferred_element_type=jnp.float32) * d1
    hs2p[...] = jnp.dot(h2, W22[...], preferred_element_type=jnp.float32) * d2


def _row_spec(w):
    return pl.BlockSpec((BM, w), lambda i: (i, 0))


def _full_spec(a, b):
    return pl.BlockSpec((a, b), lambda i: (0, 0))


_t2 = pl.pallas_call(
    _t2_body,
    grid=(N_PAD // BM,),
    in_specs=[
        _row_spec(D), _row_spec(D), _row_spec(D), _row_spec(D),
        _row_spec(1), _row_spec(1),
        _full_spec(1, D), _full_spec(1, D),
        _full_spec(D, D), _full_spec(D, D), _full_spec(1, D),
        _full_spec(D, D), _full_spec(1, D),
        _full_spec(D, D), _full_spec(D, D),
    ],
    out_specs=[_row_spec(D)] * 2,
    out_shape=[jax.ShapeDtypeStruct((N_PAD, D), jnp.float32)] * 2,
)


def _t3_body(agg1, agg2, hs1, hs2, deg1, deg2, b21, b22,
             M2a1, M2a2, m2ab, M2b, m2bb, hfin):
    i = pl.program_id(0)
    d1 = lax.rsqrt(deg1[...] + 1.0)
    d2 = lax.rsqrt(deg2[...] + 1.0)
    x1 = jax.nn.relu(d1 * (agg1[...] + hs1[...]) + b21[...])
    x2 = jax.nn.relu(d2 * (agg2[...] + hs2[...]) + b22[...])
    t = jax.nn.relu(
        jnp.dot(x1, M2a1[...], preferred_element_type=jnp.float32)
        + jnp.dot(x2, M2a2[...], preferred_element_type=jnp.float32)
        + m2ab[...])
    h = jnp.dot(t, M2b[...], preferred_element_type=jnp.float32) + m2bb[...]
    row = lax.broadcasted_iota(jnp.int32, (BM, D), 0) + i * BM
    hfin[...] = jnp.where(row < N, h, 0.0)


_t3 = pl.pallas_call(
    _t3_body,
    grid=(N_PAD // BM,),
    in_specs=[
        _row_spec(D), _row_spec(D), _row_spec(D), _row_spec(D),
        _row_spec(1), _row_spec(1),
        _full_spec(1, D), _full_spec(1, D),
        _full_spec(D, D), _full_spec(D, D), _full_spec(1, D),
        _full_spec(D, D), _full_spec(1, D),
    ],
    out_specs=_row_spec(D),
    out_shape=jax.ShapeDtypeStruct((N_PAD, D), jnp.float32),
)

LG = 128


def _t4_body(S1, S2, cnt1, cnt2, Ma1, Ma2, mab, Mb_p, mbb_p, out):
    g1 = S1[...] / jnp.maximum(cnt1[...], 1.0)
    g2 = S2[...] / jnp.maximum(cnt2[...], 1.0)
    t = jax.nn.relu(
        jnp.dot(g1, Ma1[...], preferred_element_type=jnp.float32)
        + jnp.dot(g2, Ma2[...], preferred_element_type=jnp.float32)
        + mab[...])
    lg = jnp.dot(t, Mb_p[...], preferred_element_type=jnp.float32) + mbb_p[...]
    mx = jnp.max(lg, axis=1, keepdims=True)
    e = jnp.exp(lg - mx)
    out[...] = lg - mx - jnp.log(jnp.sum(e, axis=1, keepdims=True))


_t4 = pl.pallas_call(
    _t4_body,
    grid=(1,),
    in_specs=[
        pl.BlockSpec((NUM_SEG, D), lambda i: (0, 0)),
        pl.BlockSpec((NUM_SEG, D), lambda i: (0, 0)),
        pl.BlockSpec((NUM_SEG, 1), lambda i: (0, 0)),
        pl.BlockSpec((NUM_SEG, 1), lambda i: (0, 0)),
        _full_spec(D, D), _full_spec(D, D), _full_spec(1, D),
        _full_spec(D, LG), _full_spec(1, LG),
    ],
    out_specs=pl.BlockSpec((NUM_SEG, LG), lambda i: (0, 0)),
    out_shape=jax.ShapeDtypeStruct((NUM_SEG, LG), jnp.float32),
)


def kernel(x, edge_index_1, edge_index_2, index_1, index_2,
           W11, b11, W12, b12, M1a, m1ab, M1b, m1bb,
           W21, b21, W22, b22, M2a, m2ab, M2b, m2bb, Ma, mab, Mb, mbb):
    f32 = jnp.float32
    W_pad = (jnp.zeros((K_PAD, 2 * D), f32)
             .at[:F_IN, :D].set(W11).at[:F_IN, D:].set(W12))

    pad_e = N + jnp.arange(E2R * 128 - E, dtype=jnp.int32) % (N_PAD - N)

    def edges2d(e):
        return jnp.concatenate([e, pad_e]).reshape(E2R, 128)

    src1 = edges2d(edge_index_1[0])
    dst1 = edges2d(edge_index_1[1])
    src2 = edges2d(edge_index_2[0])
    dst2 = edges2d(edge_index_2[1])
    pad_i = (NUM_SEG
             + jnp.arange(NIR * 128 - N, dtype=jnp.int32) % (SEG_PAD - NUM_SEG))
    i1 = jnp.concatenate([index_1, pad_i]).reshape(NIR, 128)
    i2 = jnp.concatenate([index_2, pad_i]).reshape(NIR, 128)

    ones128 = jnp.ones((128,), f32)
    zN = jnp.zeros((N_PAD,), f32)
    zS = jnp.zeros((SEG_PAD,), f32)
    z64 = jnp.zeros((N_PAD, D), f32)
    zS64 = jnp.zeros((SEG_PAD, D), f32)

    deg1, deg2, cnt1, cnt2 = _hist(dst1, dst2, i1, i2, ones128, zN, zS)
    deg1c = deg1[:N].reshape(N, 1)
    deg2c = deg2[:N].reshape(N, 1)

    h1, h2 = _t1(x, W_pad)
    hs1, hs2 = _t1b(h1, h2, deg1c, deg2c)
    agg1, agg2 = _mp(hs1, hs2, src1, dst1, src2, dst2, z64)
    hs1p, hs2p = _t2(agg1, agg2, hs1, hs2, deg1c, deg2c,
                     b11.reshape(1, D), b12.reshape(1, D),
                     M1a[:D], M1a[D:], m1ab.reshape(1, D),
                     M1b, m1bb.reshape(1, D), W21, W22)
    agg1p, agg2p = _mp(hs1p, hs2p, src1, dst1, src2, dst2, z64)
    hfin = _t3(agg1p, agg2p, hs1p, hs2p, deg1c, deg2c,
               b21.reshape(1, D), b22.reshape(1, D),
               M2a[:D], M2a[D:], m2ab.reshape(1, D),
               M2b, m2bb.reshape(1, D))
    S1, S2 = _pool(hfin, i1, i2, zS64)

    Mb_p = jnp.zeros((D, LG), f32).at[:, :NUM_CLASSES].set(Mb)
    mbb_p = jnp.full((1, LG), -1e30, f32).at[0, :NUM_CLASSES].set(mbb)
    out = _t4(S1[:NUM_SEG], S2[:NUM_SEG],
              cnt1[:NUM_SEG].reshape(NUM_SEG, 1),
              cnt2[:NUM_SEG].reshape(NUM_SEG, 1),
              Ma[:D], Ma[D:], mab.reshape(1, D), Mb_p, mbb_p)
    return out[:, :NUM_CLASSES]


# speedup vs baseline: 1.6003x; 1.0696x over previous
import jax
import jax.numpy as jnp
from jax import lax
from jax.experimental import pallas as pl
from jax.experimental.pallas import tpu as pltpu
from jax.experimental.pallas import tpu_sc as plsc

N = 10000
E = 320000
F_IN = 7409
D = 64
NUM_SEG = 2000
NUM_CLASSES = 7

BK = 1024
K_STEPS = 8
K_PAD = BK * K_STEPS
ER = E // 128
NR = 79
N_PAD = NR * 128
RPT = N_PAD // 16
SEG_PAD = 2048
SPT = SEG_PAD // 16
NSUP = 20
E2R = (NSUP + 1) * 128
NIR = 128

_MESH = plsc.VectorSubcoreMesh(core_axis_name="c", subcore_axis_name="s")


def _hist_body(dst1, dst2, i1, i2, ones_h, zN, zS,
               deg1_o, deg2_o, cnt1_o, cnt2_o,
               ones_v, idx0, idx1, deg_sh, cnt_sh, sem_i, sem_s):
    cid = lax.axis_index("c")
    sid = lax.axis_index("s")

    @pl.when(sid == 0)
    def _():
        pltpu.sync_copy(zN, deg_sh)
        pltpu.sync_copy(zS, cnt_sh)

    pltpu.sync_copy(ones_h, ones_v)
    plsc.subcore_barrier()

    def scatter_ones(idx2d, nsup, acc_sh):
        pltpu.sync_copy(idx2d.at[pl.ds(sid * 8, 8), :], idx0)

        def sup(u, cur, nxt):
            h_i = pltpu.async_copy(
                idx2d.at[pl.ds((u + 1) * 128 + sid * 8, 8), :], nxt, sem_i)
            hs_ = [pltpu.async_copy(ones_v, acc_sh.at[cur.at[j]], sem_s,
                                    add=True) for j in range(8)]
            for h in hs_:
                h.wait()
            h_i.wait()

        def pair(p, c):
            sup(2 * p, idx0, idx1)
            sup(2 * p + 1, idx1, idx0)
            return c

        lax.fori_loop(0, nsup // 2, pair, 0)

    def scatter_ones_once(idx2d, acc_sh):
        pltpu.sync_copy(idx2d.at[pl.ds(sid * 8, 8), :], idx0)
        hs_ = [pltpu.async_copy(ones_v, acc_sh.at[idx0.at[j]], sem_s,
                                add=True) for j in range(8)]
        for h in hs_:
            h.wait()

    @pl.when(cid == 0)
    def _():
        scatter_ones(dst1, NSUP, deg_sh)
        scatter_ones_once(i1, cnt_sh)

    @pl.when(cid == 1)
    def _():
        scatter_ones(dst2, NSUP, deg_sh)
        scatter_ones_once(i2, cnt_sh)

    plsc.subcore_barrier()

    @pl.when(jnp.logical_and(sid == 0, cid == 0))
    def _():
        pltpu.sync_copy(deg_sh, deg1_o)
        pltpu.sync_copy(cnt_sh, cnt1_o)

    @pl.when(jnp.logical_and(sid == 0, cid == 1))
    def _():
        pltpu.sync_copy(deg_sh, deg2_o)
        pltpu.sync_copy(cnt_sh, cnt2_o)


_hist = pl.kernel(
    _hist_body,
    out_type=[
        jax.ShapeDtypeStruct((N_PAD,), jnp.float32),
        jax.ShapeDtypeStruct((N_PAD,), jnp.float32),
        jax.ShapeDtypeStruct((SEG_PAD,), jnp.float32),
        jax.ShapeDtypeStruct((SEG_PAD,), jnp.float32),
    ],
    mesh=_MESH,
    compiler_params=pltpu.CompilerParams(use_tc_tiling_on_sc=False),
    scratch_types=[
        pltpu.VMEM((128,), jnp.float32),
        pltpu.VMEM((8, 128), jnp.int32),
        pltpu.VMEM((8, 128), jnp.int32),
        pltpu.VMEM_SHARED((N_PAD,), jnp.float32),
        pltpu.VMEM_SHARED((SEG_PAD,), jnp.float32),
        pltpu.SemaphoreType.DMA,
        pltpu.SemaphoreType.DMA,
    ],
)


def _mp_body(hs1, hs2, src1, dst1, src2, dst2, z64,
             agg1_o, agg2_o,
             sidx0, didx0, rows0, sidx1, didx1, rows1,
             acc_sh, tab_sh, sem_i, sem_g, sem_s):
    cid = lax.axis_index("c")
    sid = lax.axis_index("s")
    r0 = sid * RPT
    pltpu.sync_copy(z64.at[pl.ds(r0, RPT), :], acc_sh.at[pl.ds(r0, RPT), :])

    @pl.when(cid == 0)
    def _():
        pltpu.sync_copy(hs1.at[pl.ds(r0, RPT), :], tab_sh.at[pl.ds(r0, RPT), :])

    @pl.when(cid == 1)
    def _():
        pltpu.sync_copy(hs2.at[pl.ds(r0, RPT), :], tab_sh.at[pl.ds(r0, RPT), :])

    plsc.subcore_barrier()

    def edge_pass(src2d, dst2d):
        rows = (rows0, rows1)
        pltpu.sync_copy(src2d.at[pl.ds(sid * 8, 8), :], sidx0)
        pltpu.sync_copy(dst2d.at[pl.ds(sid * 8, 8), :], didx0)
        pltpu.async_copy(tab_sh.at[sidx0.at[0]], rows0, sem_g).wait()

        def sup(u, cur, nxt):
            sidx_c, didx_c = cur
            sidx_n, didx_n = nxt
            nbase = (u + 1) * 128 + sid * 8
            h_i1 = pltpu.async_copy(src2d.at[pl.ds(nbase, 8), :], sidx_n,
                                    sem_i)
            h_i2 = pltpu.async_copy(dst2d.at[pl.ds(nbase, 8), :], didx_n,
                                    sem_i)
            for j in range(8):
                rc = rows[j % 2]
                rn = rows[1 - j % 2]
                h_s = pltpu.async_copy(rc, acc_sh.at[didx_c.at[j]], sem_s,
                                       add=True)
                if j < 7:
                    pltpu.async_copy(tab_sh.at[sidx_c.at[j + 1]], rn,
                                     sem_g).wait()
                else:
                    h_i1.wait()
                    h_i2.wait()
                    pltpu.async_copy(tab_sh.at[sidx_n.at[0]], rn, sem_g).wait()
                h_s.wait()

        def pair(p, c):
            sup(2 * p, (sidx0, didx0), (sidx1, didx1))
            sup(2 * p + 1, (sidx1, didx1), (sidx0, didx0))
            return c

        lax.fori_loop(0, NSUP // 2, pair, 0)

    @pl.when(cid == 0)
    def _():
        edge_pass(src1, dst1)

    @pl.when(cid == 1)
    def _():
        edge_pass(src2, dst2)

    plsc.subcore_barrier()

    @pl.when(cid == 0)
    def _():
        pltpu.sync_copy(acc_sh.at[pl.ds(r0, RPT), :],
                        agg1_o.at[pl.ds(r0, RPT), :])

    @pl.when(cid == 1)
    def _():
        pltpu.sync_copy(acc_sh.at[pl.ds(r0, RPT), :],
                        agg2_o.at[pl.ds(r0, RPT), :])


_mp = pl.kernel(
    _mp_body,
    out_type=[jax.ShapeDtypeStruct((N_PAD, D), jnp.float32)] * 2,
    mesh=_MESH,
    compiler_params=pltpu.CompilerParams(use_tc_tiling_on_sc=False),
    scratch_types=[
        pltpu.VMEM((8, 128), jnp.int32),
        pltpu.VMEM((8, 128), jnp.int32),
        pltpu.VMEM((128, D), jnp.float32),
        pltpu.VMEM((8, 128), jnp.int32),
        pltpu.VMEM((8, 128), jnp.int32),
        pltpu.VMEM((128, D), jnp.float32),
        pltpu.VMEM_SHARED((N_PAD, D), jnp.float32),
        pltpu.VMEM_SHARED((N_PAD, D), jnp.float32),
        pltpu.SemaphoreType.DMA,
        pltpu.SemaphoreType.DMA,
        pltpu.SemaphoreType.DMA,
    ],
)


def _pool_body(hfin, i1, i2, zS64, s1_o, s2_o, cidx_v, rows_v, acc_sh,
               sem_g, sem_s):
    cid = lax.axis_index("c")
    sid = lax.axis_index("s")
    r0 = sid * SPT
    pltpu.sync_copy(zS64.at[pl.ds(r0, SPT), :], acc_sh.at[pl.ds(r0, SPT), :])
    plsc.subcore_barrier()

    def pool_pass(idx2d):
        pltpu.sync_copy(idx2d.at[pl.ds(sid * 8, 8), :], cidx_v)
        for j in range(8):
            row = sid * 8 + j

            @pl.when(row < NR)
            def _():
                pltpu.async_copy(hfin.at[pl.ds(row * 128, 128), :], rows_v,
                                 sem_g).wait()
                pltpu.async_copy(rows_v, acc_sh.at[cidx_v.at[j]], sem_s,
                                 add=True).wait()

    @pl.when(cid == 0)
    def _():
        pool_pass(i1)

    @pl.when(cid == 1)
    def _():
        pool_pass(i2)

    plsc.subcore_barrier()

    @pl.when(cid == 0)
    def _():
        pltpu.sync_copy(acc_sh.at[pl.ds(r0, SPT), :],
                        s1_o.at[pl.ds(r0, SPT), :])

    @pl.when(cid == 1)
    def _():
        pltpu.sync_copy(acc_sh.at[pl.ds(r0, SPT), :],
                        s2_o.at[pl.ds(r0, SPT), :])


_pool = pl.kernel(
    _pool_body,
    out_type=[jax.ShapeDtypeStruct((SEG_PAD, D), jnp.float32)] * 2,
    mesh=_MESH,
    compiler_params=pltpu.CompilerParams(use_tc_tiling_on_sc=False),
    scratch_types=[
        pltpu.VMEM((8, 128), jnp.int32),
        pltpu.VMEM((128, D), jnp.float32),
        pltpu.VMEM_SHARED((SEG_PAD, D), jnp.float32),
        pltpu.SemaphoreType.DMA,
        pltpu.SemaphoreType.DMA,
    ],
)


BM = 632


def _t1_body(x_ref, w_ref, h1_ref, h2_ref, acc_ref):
    k = pl.program_id(1)

    @pl.when(k == 0)
    def _():
        acc_ref[...] = jnp.zeros_like(acc_ref)

    @pl.when(k < K_STEPS - 1)
    def _():
        acc_ref[...] += jnp.dot(x_ref[...], w_ref[...],
                                preferred_element_type=jnp.float32)

    @pl.when(k == K_STEPS - 1)
    def _():
        col = lax.broadcasted_iota(jnp.int32, (BM, BK), 1) + k * BK
        xb = jnp.where(col < F_IN, x_ref[...], 0.0)
        h = acc_ref[...] + jnp.dot(xb, w_ref[...],
                                   preferred_element_type=jnp.float32)
        h1_ref[...] = h[:, :D]
        h2_ref[...] = h[:, D:]


_t1 = pl.pallas_call(
    _t1_body,
    grid=(N_PAD // BM, K_STEPS),
    in_specs=[
        pl.BlockSpec((BM, BK), lambda i, k: (i, k)),
        pl.BlockSpec((BK, 2 * D), lambda i, k: (k, 0)),
    ],
    out_specs=[pl.BlockSpec((BM, D), lambda i, k: (i, 0))] * 2,
    out_shape=[jax.ShapeDtypeStruct((N_PAD, D), jnp.float32)] * 2,
    scratch_shapes=[pltpu.VMEM((BM, 2 * D), jnp.float32)],
    compiler_params=pltpu.CompilerParams(
        dimension_semantics=("parallel", "arbitrary")),
)


def _t1b_body(h1, h2, deg1, deg2, hs1_ref, hs2_ref):
    hs1_ref[...] = h1[...] * lax.rsqrt(deg1[...] + 1.0)
    hs2_ref[...] = h2[...] * lax.rsqrt(deg2[...] + 1.0)


_t1b = pl.pallas_call(
    _t1b_body,
    grid=(N_PAD // BM,),
    in_specs=[
        pl.BlockSpec((BM, D), lambda i: (i, 0)),
        pl.BlockSpec((BM, D), lambda i: (i, 0)),
        pl.BlockSpec((BM, 1), lambda i: (i, 0)),
        pl.BlockSpec((BM, 1), lambda i: (i, 0)),
    ],
    out_specs=[pl.BlockSpec((BM, D), lambda i: (i, 0))] * 2,
    out_shape=[jax.ShapeDtypeStruct((N_PAD, D), jnp.float32)] * 2,
)


def _t2_body(agg1, agg2, hs1, hs2, deg1, deg2, b11, b12,
             M1a1, M1a2, m1ab, M1b, m1bb, W21, W22, hs1p, hs2p):
    d1 = lax.rsqrt(deg1[...] + 1.0)
    d2 = lax.rsqrt(deg2[...] + 1.0)
    x1 = jax.nn.relu(d1 * (agg1[...] + hs1[...]) + b11[...])
    x2 = jax.nn.relu(d2 * (agg2[...] + hs2[...]) + b12[...])
    t = jax.nn.relu(
        jnp.dot(x1, M1a1[...], preferred_element_type=jnp.float32)
        + jnp.dot(x2, M1a2[...], preferred_element_type=jnp.float32)
        + m1ab[...])
    h2 = jnp.dot(t, M1b[...], preferred_element_type=jnp.float32) + m1bb[...]
    hs1p[...] = jnp.dot(h2, W21[...], preferred_element_type=jnp.float32) * d1
    hs2p[...] = jnp.dot(h2, W22[...], preferred_element_type=jnp.float32) * d2


def _row_spec(w):
    return pl.BlockSpec((BM, w), lambda i: (i, 0))


def _full_spec(a, b):
    return pl.BlockSpec((a, b), lambda i: (0, 0))


_t2 = pl.pallas_call(
    _t2_body,
    grid=(N_PAD // BM,),
    in_specs=[
        _row_spec(D), _row_spec(D), _row_spec(D), _row_spec(D),
        _row_spec(1), _row_spec(1),
        _full_spec(1, D), _full_spec(1, D),
        _full_spec(D, D), _full_spec(D, D), _full_spec(1, D),
        _full_spec(D, D), _full_spec(1, D),
        _full_spec(D, D), _full_spec(D, D),
    ],
    out_specs=[_row_spec(D)] * 2,
    out_shape=[jax.ShapeDtypeStruct((N_PAD, D), jnp.float32)] * 2,
)


def _t3_body(agg1, agg2, hs1, hs2, deg1, deg2, b21, b22,
             M2a1, M2a2, m2ab, M2b, m2bb, hfin):
    i = pl.program_id(0)
    d1 = lax.rsqrt(deg1[...] + 1.0)
    d2 = lax.rsqrt(deg2[...] + 1.0)
    x1 = jax.nn.relu(d1 * (agg1[...] + hs1[...]) + b21[...])
    x2 = jax.nn.relu(d2 * (agg2[...] + hs2[...]) + b22[...])
    t = jax.nn.relu(
        jnp.dot(x1, M2a1[...], preferred_element_type=jnp.float32)
        + jnp.dot(x2, M2a2[...], preferred_element_type=jnp.float32)
        + m2ab[...])
    h = jnp.dot(t, M2b[...], preferred_element_type=jnp.float32) + m2bb[...]
    row = lax.broadcasted_iota(jnp.int32, (BM, D), 0) + i * BM
    hfin[...] = jnp.where(row < N, h, 0.0)


_t3 = pl.pallas_call(
    _t3_body,
    grid=(N_PAD // BM,),
    in_specs=[
        _row_spec(D), _row_spec(D), _row_spec(D), _row_spec(D),
        _row_spec(1), _row_spec(1),
        _full_spec(1, D), _full_spec(1, D),
        _full_spec(D, D), _full_spec(D, D), _full_spec(1, D),
        _full_spec(D, D), _full_spec(1, D),
    ],
    out_specs=_row_spec(D),
    out_shape=jax.ShapeDtypeStruct((N_PAD, D), jnp.float32),
)

LG = 128


def _t4_body(S1, S2, cnt1, cnt2, Ma1, Ma2, mab, Mb_p, mbb_p, out):
    g1 = S1[...] / jnp.maximum(cnt1[...], 1.0)
    g2 = S2[...] / jnp.maximum(cnt2[...], 1.0)
    t = jax.nn.relu(
        jnp.dot(g1, Ma1[...], preferred_element_type=jnp.float32)
        + jnp.dot(g2, Ma2[...], preferred_element_type=jnp.float32)
        + mab[...])
    lg = jnp.dot(t, Mb_p[...], preferred_element_type=jnp.float32) + mbb_p[...]
    mx = jnp.max(lg, axis=1, keepdims=True)
    e = jnp.exp(lg - mx)
    out[...] = lg - mx - jnp.log(jnp.sum(e, axis=1, keepdims=True))


_t4 = pl.pallas_call(
    _t4_body,
    grid=(1,),
    in_specs=[
        pl.BlockSpec((NUM_SEG, D), lambda i: (0, 0)),
        pl.BlockSpec((NUM_SEG, D), lambda i: (0, 0)),
        pl.BlockSpec((NUM_SEG, 1), lambda i: (0, 0)),
        pl.BlockSpec((NUM_SEG, 1), lambda i: (0, 0)),
        _full_spec(D, D), _full_spec(D, D), _full_spec(1, D),
        _full_spec(D, LG), _full_spec(1, LG),
    ],
    out_specs=pl.BlockSpec((NUM_SEG, LG), lambda i: (0, 0)),
    out_shape=jax.ShapeDtypeStruct((NUM_SEG, LG), jnp.float32),
)


def kernel(x, edge_index_1, edge_index_2, index_1, index_2,
           W11, b11, W12, b12, M1a, m1ab, M1b, m1bb,
           W21, b21, W22, b22, M2a, m2ab, M2b, m2bb, Ma, mab, Mb, mbb):
    f32 = jnp.float32
    W_pad = (jnp.zeros((K_PAD, 2 * D), f32)
             .at[:F_IN, :D].set(W11).at[:F_IN, D:].set(W12))

    pad_e = N + jnp.arange(E2R * 128 - E, dtype=jnp.int32) % (N_PAD - N)

    def edges2d(e):
        return jnp.concatenate([e, pad_e]).reshape(E2R, 128)

    src1 = edges2d(edge_index_1[0])
    dst1 = edges2d(edge_index_1[1])
    src2 = edges2d(edge_index_2[0])
    dst2 = edges2d(edge_index_2[1])
    pad_i = (NUM_SEG
             + jnp.arange(NIR * 128 - N, dtype=jnp.int32) % (SEG_PAD - NUM_SEG))
    i1 = jnp.concatenate([index_1, pad_i]).reshape(NIR, 128)
    i2 = jnp.concatenate([index_2, pad_i]).reshape(NIR, 128)

    ones128 = jnp.ones((128,), f32)
    zN = jnp.zeros((N_PAD,), f32)
    zS = jnp.zeros((SEG_PAD,), f32)
    z64 = jnp.zeros((N_PAD, D), f32)
    zS64 = jnp.zeros((SEG_PAD, D), f32)

    deg1, deg2, cnt1, cnt2 = _hist(dst1, dst2, i1, i2, ones128, zN, zS)
    deg1c = deg1[:N].reshape(N, 1)
    deg2c = deg2[:N].reshape(N, 1)

    h1, h2 = _t1(x, W_pad)
    hs1, hs2 = _t1b(h1, h2, deg1c, deg2c)
    agg1, agg2 = _mp(hs1, hs2, src1, dst1, src2, dst2, z64)
    hs1p, hs2p = _t2(agg1, agg2, hs1, hs2, deg1c, deg2c,
                     b11.reshape(1, D), b12.reshape(1, D),
                     M1a[:D], M1a[D:], m1ab.reshape(1, D),
                     M1b, m1bb.reshape(1, D), W21, W22)
    agg1p, agg2p = _mp(hs1p, hs2p, src1, dst1, src2, dst2, z64)
    hfin = _t3(agg1p, agg2p, hs1p, hs2p, deg1c, deg2c,
               b21.reshape(1, D), b22.reshape(1, D),
               M2a[:D], M2a[D:], m2ab.reshape(1, D),
               M2b, m2bb.reshape(1, D))
    S1, S2 = _pool(hfin, i1, i2, zS64)

    Mb_p = jnp.zeros((D, LG), f32).at[:, :NUM_CLASSES].set(Mb)
    mbb_p = jnp.full((1, LG), -1e30, f32).at[0, :NUM_CLASSES].set(mbb)
    out = _t4(S1[:NUM_SEG], S2[:NUM_SEG],
              cnt1[:NUM_SEG].reshape(NUM_SEG, 1),
              cnt2[:NUM_SEG].reshape(NUM_SEG, 1),
              Ma[:D], Ma[D:], mab.reshape(1, D), Mb_p, mbb_p)
    return out[:, :NUM_CLASSES]


# speedup vs baseline: 1.6836x; 1.0520x over previous
import jax
import jax.numpy as jnp
from jax import lax
from jax.experimental import pallas as pl
from jax.experimental.pallas import tpu as pltpu
from jax.experimental.pallas import tpu_sc as plsc

N = 10000
E = 320000
F_IN = 7409
D = 64
NUM_SEG = 2000
NUM_CLASSES = 7

BK = 1024
K_STEPS = 8
K_PAD = BK * K_STEPS
ER = E // 128
NR = 79
N_PAD = NR * 128
RPT = N_PAD // 16
SEG_PAD = 2048
SPT = SEG_PAD // 16
NSUP = 20
E2R = (NSUP + 1) * 128
NIR = 128

_MESH = plsc.VectorSubcoreMesh(core_axis_name="c", subcore_axis_name="s")


def _hist_body(dst1, dst2, i1, i2, ones_h, zN, zS,
               deg1_o, deg2_o, cnt1_o, cnt2_o,
               ones_v, idx0, idx1, deg_sh, cnt_sh, sem_i, sem_s):
    cid = lax.axis_index("c")
    sid = lax.axis_index("s")

    @pl.when(sid == 0)
    def _():
        pltpu.sync_copy(zN, deg_sh)
        pltpu.sync_copy(zS, cnt_sh)

    pltpu.sync_copy(ones_h, ones_v)
    plsc.subcore_barrier()

    def scatter_ones(idx2d, nsup, acc_sh):
        pltpu.sync_copy(idx2d.at[pl.ds(sid * 8, 8), :], idx0)

        def sup(u, cur, nxt):
            h_i = pltpu.async_copy(
                idx2d.at[pl.ds((u + 1) * 128 + sid * 8, 8), :], nxt, sem_i)
            hs_ = [pltpu.async_copy(ones_v, acc_sh.at[cur.at[j]], sem_s,
                                    add=True) for j in range(8)]
            for h in hs_:
                h.wait()
            h_i.wait()

        def pair(p, c):
            sup(2 * p, idx0, idx1)
            sup(2 * p + 1, idx1, idx0)
            return c

        lax.fori_loop(0, nsup // 2, pair, 0)

    def scatter_ones_once(idx2d, acc_sh):
        pltpu.sync_copy(idx2d.at[pl.ds(sid * 8, 8), :], idx0)
        hs_ = [pltpu.async_copy(ones_v, acc_sh.at[idx0.at[j]], sem_s,
                                add=True) for j in range(8)]
        for h in hs_:
            h.wait()

    @pl.when(cid == 0)
    def _():
        scatter_ones(dst1, NSUP, deg_sh)
        scatter_ones_once(i1, cnt_sh)

    @pl.when(cid == 1)
    def _():
        scatter_ones(dst2, NSUP, deg_sh)
        scatter_ones_once(i2, cnt_sh)

    plsc.subcore_barrier()

    @pl.when(jnp.logical_and(sid == 0, cid == 0))
    def _():
        pltpu.sync_copy(deg_sh, deg1_o)
        pltpu.sync_copy(cnt_sh, cnt1_o)

    @pl.when(jnp.logical_and(sid == 0, cid == 1))
    def _():
        pltpu.sync_copy(deg_sh, deg2_o)
        pltpu.sync_copy(cnt_sh, cnt2_o)


_hist = pl.kernel(
    _hist_body,
    out_type=[
        jax.ShapeDtypeStruct((N_PAD,), jnp.float32),
        jax.ShapeDtypeStruct((N_PAD,), jnp.float32),
        jax.ShapeDtypeStruct((SEG_PAD,), jnp.float32),
        jax.ShapeDtypeStruct((SEG_PAD,), jnp.float32),
    ],
    mesh=_MESH,
    compiler_params=pltpu.CompilerParams(use_tc_tiling_on_sc=False),
    scratch_types=[
        pltpu.VMEM((128,), jnp.float32),
        pltpu.VMEM((8, 128), jnp.int32),
        pltpu.VMEM((8, 128), jnp.int32),
        pltpu.VMEM_SHARED((N_PAD,), jnp.float32),
        pltpu.VMEM_SHARED((SEG_PAD,), jnp.float32),
        pltpu.SemaphoreType.DMA,
        pltpu.SemaphoreType.DMA,
    ],
)


def _mp_body(hs1, hs2, src1, dst1, src2, dst2, z64,
             agg1_o, agg2_o,
             sidx0, didx0, rows0, sidx1, didx1, rows1, rows2, rows3,
             acc_sh, tab_sh, sem_i, sem_g, sem_s):
    cid = lax.axis_index("c")
    sid = lax.axis_index("s")
    r0 = sid * RPT
    pltpu.sync_copy(z64.at[pl.ds(r0, RPT), :], acc_sh.at[pl.ds(r0, RPT), :])

    @pl.when(cid == 0)
    def _():
        pltpu.sync_copy(hs1.at[pl.ds(r0, RPT), :], tab_sh.at[pl.ds(r0, RPT), :])

    @pl.when(cid == 1)
    def _():
        pltpu.sync_copy(hs2.at[pl.ds(r0, RPT), :], tab_sh.at[pl.ds(r0, RPT), :])

    plsc.subcore_barrier()

    def edge_pass(src2d, dst2d):
        rows = (rows0, rows1, rows2, rows3)
        pltpu.sync_copy(src2d.at[pl.ds(sid * 8, 8), :], sidx0)
        pltpu.sync_copy(dst2d.at[pl.ds(sid * 8, 8), :], didx0)
        pltpu.async_copy(tab_sh.at[sidx0.at[0]], rows0, sem_g).wait()

        def sup(u, cur, nxt):
            sidx_c, didx_c = cur
            sidx_n, didx_n = nxt
            nbase = (u + 1) * 128 + sid * 8
            h_i1 = pltpu.async_copy(src2d.at[pl.ds(nbase, 8), :], sidx_n,
                                    sem_i)
            h_i2 = pltpu.async_copy(dst2d.at[pl.ds(nbase, 8), :], didx_n,
                                    sem_i)
            hss = []
            for j in range(8):
                hss.append(pltpu.async_copy(rows[j % 4],
                                            acc_sh.at[didx_c.at[j]], sem_s,
                                            add=True))
                if j >= 3:
                    hss[j - 3].wait()
                rn = rows[(j + 1) % 4]
                if j < 7:
                    pltpu.async_copy(tab_sh.at[sidx_c.at[j + 1]], rn,
                                     sem_g).wait()
                else:
                    h_i1.wait()
                    h_i2.wait()
                    pltpu.async_copy(tab_sh.at[sidx_n.at[0]], rn, sem_g).wait()
            hss[5].wait()
            hss[6].wait()
            hss[7].wait()

        def pair(p, c):
            sup(2 * p, (sidx0, didx0), (sidx1, didx1))
            sup(2 * p + 1, (sidx1, didx1), (sidx0, didx0))
            return c

        lax.fori_loop(0, NSUP // 2, pair, 0)

    @pl.when(cid == 0)
    def _():
        edge_pass(src1, dst1)

    @pl.when(cid == 1)
    def _():
        edge_pass(src2, dst2)

    plsc.subcore_barrier()

    @pl.when(cid == 0)
    def _():
        pltpu.sync_copy(acc_sh.at[pl.ds(r0, RPT), :],
                        agg1_o.at[pl.ds(r0, RPT), :])

    @pl.when(cid == 1)
    def _():
        pltpu.sync_copy(acc_sh.at[pl.ds(r0, RPT), :],
                        agg2_o.at[pl.ds(r0, RPT), :])


_mp = pl.kernel(
    _mp_body,
    out_type=[jax.ShapeDtypeStruct((N_PAD, D), jnp.float32)] * 2,
    mesh=_MESH,
    compiler_params=pltpu.CompilerParams(use_tc_tiling_on_sc=False),
    scratch_types=[
        pltpu.VMEM((8, 128), jnp.int32),
        pltpu.VMEM((8, 128), jnp.int32),
        pltpu.VMEM((128, D), jnp.float32),
        pltpu.VMEM((8, 128), jnp.int32),
        pltpu.VMEM((8, 128), jnp.int32),
        pltpu.VMEM((128, D), jnp.float32),
        pltpu.VMEM((128, D), jnp.float32),
        pltpu.VMEM((128, D), jnp.float32),
        pltpu.VMEM_SHARED((N_PAD, D), jnp.float32),
        pltpu.VMEM_SHARED((N_PAD, D), jnp.float32),
        pltpu.SemaphoreType.DMA,
        pltpu.SemaphoreType.DMA,
        pltpu.SemaphoreType.DMA,
    ],
)


def _pool_body(hfin, i1, i2, zS64, s1_o, s2_o, cidx_v, rows_v, acc_sh,
               sem_g, sem_s):
    cid = lax.axis_index("c")
    sid = lax.axis_index("s")
    r0 = sid * SPT
    pltpu.sync_copy(zS64.at[pl.ds(r0, SPT), :], acc_sh.at[pl.ds(r0, SPT), :])
    plsc.subcore_barrier()

    def pool_pass(idx2d):
        pltpu.sync_copy(idx2d.at[pl.ds(sid * 8, 8), :], cidx_v)
        for j in range(8):
            row = sid * 8 + j

            @pl.when(row < NR)
            def _():
                pltpu.async_copy(hfin.at[pl.ds(row * 128, 128), :], rows_v,
                                 sem_g).wait()
                pltpu.async_copy(rows_v, acc_sh.at[cidx_v.at[j]], sem_s,
                                 add=True).wait()

    @pl.when(cid == 0)
    def _():
        pool_pass(i1)

    @pl.when(cid == 1)
    def _():
        pool_pass(i2)

    plsc.subcore_barrier()

    @pl.when(cid == 0)
    def _():
        pltpu.sync_copy(acc_sh.at[pl.ds(r0, SPT), :],
                        s1_o.at[pl.ds(r0, SPT), :])

    @pl.when(cid == 1)
    def _():
        pltpu.sync_copy(acc_sh.at[pl.ds(r0, SPT), :],
                        s2_o.at[pl.ds(r0, SPT), :])


_pool = pl.kernel(
    _pool_body,
    out_type=[jax.ShapeDtypeStruct((SEG_PAD, D), jnp.float32)] * 2,
    mesh=_MESH,
    compiler_params=pltpu.CompilerParams(use_tc_tiling_on_sc=False),
    scratch_types=[
        pltpu.VMEM((8, 128), jnp.int32),
        pltpu.VMEM((128, D), jnp.float32),
        pltpu.VMEM_SHARED((SEG_PAD, D), jnp.float32),
        pltpu.SemaphoreType.DMA,
        pltpu.SemaphoreType.DMA,
    ],
)


BM = 632


def _t1_body(x_ref, w_ref, h1_ref, h2_ref, acc_ref):
    k = pl.program_id(1)

    @pl.when(k == 0)
    def _():
        acc_ref[...] = jnp.zeros_like(acc_ref)

    wb = w_ref[pl.ds(k * BK, BK), :]

    @pl.when(k < K_STEPS - 1)
    def _():
        acc_ref[...] += jnp.dot(x_ref[...], wb,
                                preferred_element_type=jnp.float32)

    @pl.when(k == K_STEPS - 1)
    def _():
        col = lax.broadcasted_iota(jnp.int32, (BM, BK), 1) + k * BK
        xb = jnp.where(col < F_IN, x_ref[...], 0.0)
        h = acc_ref[...] + jnp.dot(xb, wb,
                                   preferred_element_type=jnp.float32)
        h1_ref[...] = h[:, :D]
        h2_ref[...] = h[:, D:]


_t1 = pl.pallas_call(
    _t1_body,
    grid=(N_PAD // BM, K_STEPS),
    in_specs=[
        pl.BlockSpec((BM, BK), lambda i, k: (i, k)),
        pl.BlockSpec((K_PAD, 2 * D), lambda i, k: (0, 0)),
    ],
    out_specs=[pl.BlockSpec((BM, D), lambda i, k: (i, 0))] * 2,
    out_shape=[jax.ShapeDtypeStruct((N_PAD, D), jnp.float32)] * 2,
    scratch_shapes=[pltpu.VMEM((BM, 2 * D), jnp.float32)],
    compiler_params=pltpu.CompilerParams(
        dimension_semantics=("parallel", "arbitrary")),
)


def _t1b_body(h1, h2, deg1, deg2, hs1_ref, hs2_ref):
    hs1_ref[...] = h1[...] * lax.rsqrt(deg1[...] + 1.0)
    hs2_ref[...] = h2[...] * lax.rsqrt(deg2[...] + 1.0)


_t1b = pl.pallas_call(
    _t1b_body,
    grid=(N_PAD // BM,),
    in_specs=[
        pl.BlockSpec((BM, D), lambda i: (i, 0)),
        pl.BlockSpec((BM, D), lambda i: (i, 0)),
        pl.BlockSpec((BM, 1), lambda i: (i, 0)),
        pl.BlockSpec((BM, 1), lambda i: (i, 0)),
    ],
    out_specs=[pl.BlockSpec((BM, D), lambda i: (i, 0))] * 2,
    out_shape=[jax.ShapeDtypeStruct((N_PAD, D), jnp.float32)] * 2,
)


def _t2_body(agg1, agg2, hs1, hs2, deg1, deg2, b11, b12,
             M1a1, M1a2, m1ab, M1b, m1bb, W21, W22, hs1p, hs2p):
    d1 = lax.rsqrt(deg1[...] + 1.0)
    d2 = lax.rsqrt(deg2[...] + 1.0)
    x1 = jax.nn.relu(d1 * (agg1[...] + hs1[...]) + b11[...])
    x2 = jax.nn.relu(d2 * (agg2[...] + hs2[...]) + b12[...])
    t = jax.nn.relu(
        jnp.dot(x1, M1a1[...], preferred_element_type=jnp.float32)
        + jnp.dot(x2, M1a2[...], preferred_element_type=jnp.float32)
        + m1ab[...])
    h2 = jnp.dot(t, M1b[...], preferred_element_type=jnp.float32) + m1bb[...]
    hs1p[...] = jnp.dot(h2, W21[...], preferred_element_type=jnp.float32) * d1
    hs2p[...] = jnp.dot(h2, W22[...], preferred_element_type=jnp.float32) * d2


def _row_spec(w):
    return pl.BlockSpec((BM, w), lambda i: (i, 0))


def _full_spec(a, b):
    return pl.BlockSpec((a, b), lambda i: (0, 0))


_t2 = pl.pallas_call(
    _t2_body,
    grid=(N_PAD // BM,),
    in_specs=[
        _row_spec(D), _row_spec(D), _row_spec(D), _row_spec(D),
        _row_spec(1), _row_spec(1),
        _full_spec(1, D), _full_spec(1, D),
        _full_spec(D, D), _full_spec(D, D), _full_spec(1, D),
        _full_spec(D, D), _full_spec(1, D),
        _full_spec(D, D), _full_spec(D, D),
    ],
    out_specs=[_row_spec(D)] * 2,
    out_shape=[jax.ShapeDtypeStruct((N_PAD, D), jnp.float32)] * 2,
)


def _t3_body(agg1, agg2, hs1, hs2, deg1, deg2, b21, b22,
             M2a1, M2a2, m2ab, M2b, m2bb, hfin):
    i = pl.program_id(0)
    d1 = lax.rsqrt(deg1[...] + 1.0)
    d2 = lax.rsqrt(deg2[...] + 1.0)
    x1 = jax.nn.relu(d1 * (agg1[...] + hs1[...]) + b21[...])
    x2 = jax.nn.relu(d2 * (agg2[...] + hs2[...]) + b22[...])
    t = jax.nn.relu(
        jnp.dot(x1, M2a1[...], preferred_element_type=jnp.float32)
        + jnp.dot(x2, M2a2[...], preferred_element_type=jnp.float32)
        + m2ab[...])
    h = jnp.dot(t, M2b[...], preferred_element_type=jnp.float32) + m2bb[...]
    row = lax.broadcasted_iota(jnp.int32, (BM, D), 0) + i * BM
    hfin[...] = jnp.where(row < N, h, 0.0)


_t3 = pl.pallas_call(
    _t3_body,
    grid=(N_PAD // BM,),
    in_specs=[
        _row_spec(D), _row_spec(D), _row_spec(D), _row_spec(D),
        _row_spec(1), _row_spec(1),
        _full_spec(1, D), _full_spec(1, D),
        _full_spec(D, D), _full_spec(D, D), _full_spec(1, D),
        _full_spec(D, D), _full_spec(1, D),
    ],
    out_specs=_row_spec(D),
    out_shape=jax.ShapeDtypeStruct((N_PAD, D), jnp.float32),
)

LG = 128


def _t4_body(S1, S2, cnt1, cnt2, Ma1, Ma2, mab, Mb_p, mbb_p, out):
    g1 = S1[...] / jnp.maximum(cnt1[...], 1.0)
    g2 = S2[...] / jnp.maximum(cnt2[...], 1.0)
    t = jax.nn.relu(
        jnp.dot(g1, Ma1[...], preferred_element_type=jnp.float32)
        + jnp.dot(g2, Ma2[...], preferred_element_type=jnp.float32)
        + mab[...])
    lg = jnp.dot(t, Mb_p[...], preferred_element_type=jnp.float32) + mbb_p[...]
    mx = jnp.max(lg, axis=1, keepdims=True)
    e = jnp.exp(lg - mx)
    out[...] = lg - mx - jnp.log(jnp.sum(e, axis=1, keepdims=True))


_t4 = pl.pallas_call(
    _t4_body,
    grid=(1,),
    in_specs=[
        pl.BlockSpec((NUM_SEG, D), lambda i: (0, 0)),
        pl.BlockSpec((NUM_SEG, D), lambda i: (0, 0)),
        pl.BlockSpec((NUM_SEG, 1), lambda i: (0, 0)),
        pl.BlockSpec((NUM_SEG, 1), lambda i: (0, 0)),
        _full_spec(D, D), _full_spec(D, D), _full_spec(1, D),
        _full_spec(D, LG), _full_spec(1, LG),
    ],
    out_specs=pl.BlockSpec((NUM_SEG, LG), lambda i: (0, 0)),
    out_shape=jax.ShapeDtypeStruct((NUM_SEG, LG), jnp.float32),
)


def kernel(x, edge_index_1, edge_index_2, index_1, index_2,
           W11, b11, W12, b12, M1a, m1ab, M1b, m1bb,
           W21, b21, W22, b22, M2a, m2ab, M2b, m2bb, Ma, mab, Mb, mbb):
    f32 = jnp.float32
    W_pad = (jnp.zeros((K_PAD, 2 * D), f32)
             .at[:F_IN, :D].set(W11).at[:F_IN, D:].set(W12))

    pad_e = N + jnp.arange(E2R * 128 - E, dtype=jnp.int32) % (N_PAD - N)

    def edges2d(e):
        return jnp.concatenate([e, pad_e]).reshape(E2R, 128)

    src1 = edges2d(edge_index_1[0])
    dst1 = edges2d(edge_index_1[1])
    src2 = edges2d(edge_index_2[0])
    dst2 = edges2d(edge_index_2[1])
    pad_i = (NUM_SEG
             + jnp.arange(NIR * 128 - N, dtype=jnp.int32) % (SEG_PAD - NUM_SEG))
    i1 = jnp.concatenate([index_1, pad_i]).reshape(NIR, 128)
    i2 = jnp.concatenate([index_2, pad_i]).reshape(NIR, 128)

    ones128 = jnp.ones((128,), f32)
    zN = jnp.zeros((N_PAD,), f32)
    zS = jnp.zeros((SEG_PAD,), f32)
    z64 = jnp.zeros((N_PAD, D), f32)
    zS64 = jnp.zeros((SEG_PAD, D), f32)

    deg1, deg2, cnt1, cnt2 = _hist(dst1, dst2, i1, i2, ones128, zN, zS)
    deg1c = deg1[:N].reshape(N, 1)
    deg2c = deg2[:N].reshape(N, 1)

    h1, h2 = _t1(x, W_pad)
    hs1, hs2 = _t1b(h1, h2, deg1c, deg2c)
    agg1, agg2 = _mp(hs1, hs2, src1, dst1, src2, dst2, z64)
    hs1p, hs2p = _t2(agg1, agg2, hs1, hs2, deg1c, deg2c,
                     b11.reshape(1, D), b12.reshape(1, D),
                     M1a[:D], M1a[D:], m1ab.reshape(1, D),
                     M1b, m1bb.reshape(1, D), W21, W22)
    agg1p, agg2p = _mp(hs1p, hs2p, src1, dst1, src2, dst2, z64)
    hfin = _t3(agg1p, agg2p, hs1p, hs2p, deg1c, deg2c,
               b21.reshape(1, D), b22.reshape(1, D),
               M2a[:D], M2a[D:], m2ab.reshape(1, D),
               M2b, m2bb.reshape(1, D))
    S1, S2 = _pool(hfin, i1, i2, zS64)

    Mb_p = jnp.zeros((D, LG), f32).at[:, :NUM_CLASSES].set(Mb)
    mbb_p = jnp.full((1, LG), -1e30, f32).at[0, :NUM_CLASSES].set(mbb)
    out = _t4(S1[:NUM_SEG], S2[:NUM_SEG],
              cnt1[:NUM_SEG].reshape(NUM_SEG, 1),
              cnt2[:NUM_SEG].reshape(NUM_SEG, 1),
              Ma[:D], Ma[D:], mab.reshape(1, D), Mb_p, mbb_p)
    return out[:, :NUM_CLASSES]
